# Initial kernel scaffold; baseline (speedup 1.0000x reference)
import jax
import jax.numpy as jnp
from jax.experimental import pallas as pl


def kernel(scalar_node_features, vector_node_features, normdir, edge_index, edge_weight, edge_attr, Wf, bf, W1, b1, W2, b2):
    raise NotImplementedError("write your pallas kernel here")


# speedup vs baseline: 8.8154x; 8.8154x over previous
import functools

import jax
import jax.numpy as jnp
from jax import lax
from jax.experimental import pallas as pl
from jax.experimental.pallas import tpu as pltpu
from jax.experimental.pallas import tpu_sc as plsc

CUTOFF = 5.0
NFC = 4
FC = 32
NC, NS = 2, 16
WE = 64


def _node_tables_kernel(xs_ref, xv_ref, w1_ref, b1_ref, w2_ref, b2_ref,
                        xtab_ref, init_ref):
    xs = xs_ref[...]
    h = xs @ w1_ref[...] + b1_ref[...][None, :]
    h = h * jax.nn.sigmoid(h)
    x = h @ w2_ref[...] + b2_ref[...][None, :]
    for fc in range(NFC):
        sl = slice(fc * FC, fc * FC + FC)
        xtab_ref[fc, :, 0:32] = x[:, fc * FC:fc * FC + FC]
        xtab_ref[fc, :, 32:64] = x[:, 128 + fc * FC:128 + fc * FC + FC]
        xtab_ref[fc, :, 64:96] = x[:, 256 + fc * FC:256 + fc * FC + FC]
        init_ref[fc, :, 0:32] = xs[:, sl]
        for c in range(3):
            v = xv_ref[c, :, sl]
            xtab_ref[fc, :, 96 + 32 * c:128 + 32 * c] = v
            init_ref[fc, :, 32 + 32 * c:64 + 32 * c] = v


def _edge_filter_kernel(ea_ref, ew_ref, nd_ref, wf_ref, bf_ref, wmod_ref):
    ew = ew_ref[...]
    c = 0.5 * (jnp.cos(jnp.pi * ew / CUTOFF) + 1.0)
    c = c * (ew < CUTOFF).astype(jnp.float32)
    w = (ea_ref[...] @ wf_ref[...] + bf_ref[...][None, :]) * c
    nd = nd_ref[...]
    for fc in range(NFC):
        wmod_ref[fc, :, 0:32] = w[:, fc * FC:fc * FC + FC]
        wmod_ref[fc, :, 32:64] = w[:, 128 + fc * FC:128 + fc * FC + FC]
        wmod_ref[fc, :, 64:96] = w[:, 256 + fc * FC:256 + fc * FC + FC]
        wmod_ref[fc, :, 96:128] = nd


def _sc_scatter_body(ept, nwin,
                     xtab, wmod, srcp, dstp, init, out,
                     sidx, didx, rows, wrow, msg, acc, sem):
    c = lax.axis_index("c")
    s = lax.axis_index("s")
    for p in range(2):
        fc = c * 2 + p

        @pl.when(s == 0)
        def _():
            pltpu.sync_copy(init.at[fc], acc)

        plsc.subcore_barrier()

        tile_base = s * ept

        def win(wi, carry):
            base = tile_base + wi * WE
            pltpu.sync_copy(srcp.at[pl.ds(base, WE)], sidx)
            pltpu.sync_copy(dstp.at[pl.ds(base, WE)], didx)
            pltpu.async_copy(xtab.at[fc].at[sidx], rows, sem).wait()
            pltpu.sync_copy(wmod.at[fc].at[pl.ds(base, WE)], wrow)

            def edge(e, carry2):
                nd16 = wrow[e, pl.ds(96, 16)]
                z16 = jnp.zeros((16,), jnp.int32)
                ndx = jnp.take_along_axis(nd16, z16, axis=0)
                ndy = jnp.take_along_axis(nd16, z16 + 1, axis=0)
                ndz = jnp.take_along_axis(nd16, z16 + 2, axis=0)
                for j in range(2):
                    o = j * 16
                    x0 = rows[e, pl.ds(0 + o, 16)]
                    x1 = rows[e, pl.ds(32 + o, 16)]
                    x2 = rows[e, pl.ds(64 + o, 16)]
                    vx = rows[e, pl.ds(96 + o, 16)]
                    vy = rows[e, pl.ds(128 + o, 16)]
                    vz = rows[e, pl.ds(160 + o, 16)]
                    w0 = wrow[e, pl.ds(0 + o, 16)]
                    w1 = wrow[e, pl.ds(32 + o, 16)]
                    w2 = wrow[e, pl.ds(64 + o, 16)]
                    t = w2 * x2
                    sv = w1 * x1
                    msg[e, pl.ds(0 + o, 16)] = w0 * x0
                    msg[e, pl.ds(32 + o, 16)] = ndx * sv + t * vx
                    msg[e, pl.ds(64 + o, 16)] = ndy * sv + t * vy
                    msg[e, pl.ds(96 + o, 16)] = ndz * sv + t * vz
                return carry2

            lax.fori_loop(0, WE, edge, 0)
            pltpu.sync_copy(msg, acc.at[didx], add=True)
            return carry

        lax.fori_loop(0, nwin, win, 0)
        plsc.subcore_barrier()

        @pl.when(s == 0)
        def _():
            pltpu.sync_copy(acc, out.at[fc])

        plsc.subcore_barrier()


def kernel(scalar_node_features, vector_node_features, normdir, edge_index,
           edge_weight, edge_attr, Wf, bf, W1, b1, W2, b2):
    N, _, F = scalar_node_features.shape
    E = edge_index.shape[1]
    assert F == 128

    NP8 = N + 8
    epc = -(-E // (NS * WE)) * WE
    EP = epc * NS
    nwin = epc // WE

    xs = scalar_node_features[:, 0, :]
    xs_p = jnp.pad(xs, ((0, NP8 - N), (0, 0)))
    xv_p = jnp.pad(vector_node_features.transpose(1, 0, 2),
                   ((0, 0), (0, NP8 - N), (0, 0)))
    ea_p = jnp.pad(edge_attr, ((0, EP - E), (0, 0)))
    ew_p = jnp.pad(edge_weight, ((0, EP - E), (0, 0)))
    nd_p = jnp.pad(normdir, ((0, EP - E), (0, 29)))
    src_p = jnp.pad(edge_index[0], (0, EP - E))
    dst_p = jnp.pad(edge_index[1], (0, EP - E),
                    constant_values=N)

    nblk = 9
    nb = NP8 // nblk
    assert NP8 % nblk == 0 and nb % 8 == 0
    xtab, init = pl.pallas_call(
        _node_tables_kernel,
        grid=(nblk,),
        in_specs=[
            pl.BlockSpec((nb, F), lambda i: (i, 0)),
            pl.BlockSpec((3, nb, F), lambda i: (0, i, 0)),
            pl.BlockSpec((F, F), lambda i: (0, 0)),
            pl.BlockSpec((F,), lambda i: (0,)),
            pl.BlockSpec((F, 3 * F), lambda i: (0, 0)),
            pl.BlockSpec((3 * F,), lambda i: (0,)),
        ],
        out_specs=[
            pl.BlockSpec((NFC, nb, 256), lambda i: (0, i, 0)),
            pl.BlockSpec((NFC, nb, 128), lambda i: (0, i, 0)),
        ],
        out_shape=[
            jax.ShapeDtypeStruct((NFC, NP8, 256), jnp.float32),
            jax.ShapeDtypeStruct((NFC, NP8, 128), jnp.float32),
        ],
    )(xs_p, xv_p, W1, b1, W2, b2)

    eb = 2048
    while EP % eb:
        eb //= 2
    eblk = EP // eb
    wmod = pl.pallas_call(
        _edge_filter_kernel,
        grid=(eblk,),
        in_specs=[
            pl.BlockSpec((eb, edge_attr.shape[1]), lambda i: (i, 0)),
            pl.BlockSpec((eb, 1), lambda i: (i, 0)),
            pl.BlockSpec((eb, 32), lambda i: (i, 0)),
            pl.BlockSpec(Wf.shape, lambda i: (0, 0)),
            pl.BlockSpec(bf.shape, lambda i: (0,)),
        ],
        out_specs=pl.BlockSpec((NFC, eb, 128), lambda i: (0, i, 0)),
        out_shape=jax.ShapeDtypeStruct((NFC, EP, 128), jnp.float32),
    )(ea_p, ew_p, nd_p, Wf, bf)

    mesh = plsc.VectorSubcoreMesh(core_axis_name="c", subcore_axis_name="s",
                                  num_cores=NC, num_subcores=NS)
    out4 = pl.kernel(
        functools.partial(_sc_scatter_body, epc, nwin),
        out_type=jax.ShapeDtypeStruct((NFC, NP8, 128), jnp.float32),
        mesh=mesh,
        scratch_types=[
            pltpu.VMEM((WE,), jnp.int32),
            pltpu.VMEM((WE,), jnp.int32),
            pltpu.VMEM((WE, 256), jnp.float32),
            pltpu.VMEM((WE, 128), jnp.float32),
            pltpu.VMEM((WE, 128), jnp.float32),
            pltpu.VMEM_SHARED((NP8, 128), jnp.float32),
            pltpu.SemaphoreType.DMA,
        ],
    )(xtab, wmod, src_p, dst_p, init)

    o = out4[:, :N, :]
    q = o[:, :, 0:32].transpose(1, 0, 2).reshape(N, 1, F)
    mu = o[:, :, 32:].reshape(NFC, N, 3, FC).transpose(1, 2, 0, 3).reshape(N, 3, F)
    return (q, mu)


# speedup vs baseline: 12.4048x; 1.4072x over previous
import functools

import jax
import jax.numpy as jnp
from jax import lax
from jax.experimental import pallas as pl
from jax.experimental.pallas import tpu as pltpu
from jax.experimental.pallas import tpu_sc as plsc

CUTOFF = 5.0
NFC = 4
FC = 32
NC, NS = 2, 16
WE = 40


def _node_tables_kernel(xs_ref, xv_ref, w1_ref, b1_ref, w2_ref, b2_ref,
                        xtab_ref, init_ref):
    xs = xs_ref[...]
    h = xs @ w1_ref[...] + b1_ref[...][None, :]
    h = h * jax.nn.sigmoid(h)
    x = h @ w2_ref[...] + b2_ref[...][None, :]
    for fc in range(NFC):
        sl = slice(fc * FC, fc * FC + FC)
        xtab_ref[fc, :, 0:32] = x[:, fc * FC:fc * FC + FC]
        xtab_ref[fc, :, 32:64] = x[:, 128 + fc * FC:128 + fc * FC + FC]
        xtab_ref[fc, :, 64:96] = x[:, 256 + fc * FC:256 + fc * FC + FC]
        init_ref[fc, :, 0:32] = xs[:, sl]
        for c in range(3):
            v = xv_ref[c, :, sl]
            xtab_ref[fc, :, 96 + 32 * c:128 + 32 * c] = v
            init_ref[fc, :, 32 + 32 * c:64 + 32 * c] = v


def _edge_filter_kernel(ea_ref, ew_ref, nd_ref, wf_ref, bf_ref, wmod_ref):
    ew = ew_ref[...]
    c = 0.5 * (jnp.cos(jnp.pi * ew / CUTOFF) + 1.0)
    c = c * (ew < CUTOFF).astype(jnp.float32)
    w = (ea_ref[...] @ wf_ref[...] + bf_ref[...][None, :]) * c
    nd = nd_ref[...]
    for fc in range(NFC):
        wmod_ref[fc, :, 0:32] = w[:, fc * FC:fc * FC + FC]
        wmod_ref[fc, :, 32:64] = w[:, 128 + fc * FC:128 + fc * FC + FC]
        wmod_ref[fc, :, 64:96] = w[:, 256 + fc * FC:256 + fc * FC + FC]
        wmod_ref[fc, :, 96:128] = nd


def _sc_scatter_body(ept, nwin,
                     xtab, wmod, srcp, dstp, init, out,
                     sidx0, sidx1, didx0, didx1,
                     rows0, rows1, wrow0, wrow1, msg, acc,
                     gsem, isem):
    c = lax.axis_index("c")
    s = lax.axis_index("s")
    sidx = (sidx0, sidx1)
    didx = (didx0, didx1)
    rows = (rows0, rows1)
    wrow = (wrow0, wrow1)

    def issue_idx(w, b):
        base = s * ept + w * WE
        pltpu.async_copy(srcp.at[pl.ds(base, WE)], sidx[b], isem)
        pltpu.async_copy(dstp.at[pl.ds(base, WE)], didx[b], isem)

    def drain_idx():
        pltpu.make_async_copy(srcp.at[pl.ds(0, WE)], sidx[0], isem).wait()
        pltpu.make_async_copy(dstp.at[pl.ds(0, WE)], didx[0], isem).wait()

    def issue_main(fc, w, b):
        base = s * ept + w * WE
        pltpu.async_copy(xtab.at[fc].at[sidx[b]], rows[b], gsem)
        pltpu.async_copy(wmod.at[fc].at[pl.ds(base, WE)], wrow[b], gsem)

    def drain_main(fc):
        pltpu.make_async_copy(xtab.at[fc].at[sidx[0]], rows[0], gsem).wait()
        pltpu.make_async_copy(wmod.at[fc].at[pl.ds(0, WE)], wrow[0], gsem).wait()

    def compute(rows_b, wrow_b):
        def edge(e, carry2):
            nd16 = wrow_b[e, pl.ds(96, 16)]
            z16 = jnp.zeros((16,), jnp.int32)
            ndx = jnp.take_along_axis(nd16, z16, axis=0)
            ndy = jnp.take_along_axis(nd16, z16 + 1, axis=0)
            ndz = jnp.take_along_axis(nd16, z16 + 2, axis=0)
            for j in range(2):
                o = j * 16
                x0 = rows_b[e, pl.ds(0 + o, 16)]
                x1 = rows_b[e, pl.ds(32 + o, 16)]
                x2 = rows_b[e, pl.ds(64 + o, 16)]
                vx = rows_b[e, pl.ds(96 + o, 16)]
                vy = rows_b[e, pl.ds(128 + o, 16)]
                vz = rows_b[e, pl.ds(160 + o, 16)]
                w0 = wrow_b[e, pl.ds(0 + o, 16)]
                w1 = wrow_b[e, pl.ds(32 + o, 16)]
                w2 = wrow_b[e, pl.ds(64 + o, 16)]
                t = w2 * x2
                sv = w1 * x1
                msg[e, pl.ds(0 + o, 16)] = w0 * x0
                msg[e, pl.ds(32 + o, 16)] = ndx * sv + t * vx
                msg[e, pl.ds(64 + o, 16)] = ndy * sv + t * vy
                msg[e, pl.ds(96 + o, 16)] = ndz * sv + t * vz
            return carry2

        lax.fori_loop(0, WE, edge, 0)

    for p in range(2):
        fc = c * 2 + p

        @pl.when(s == 0)
        def _():
            pltpu.sync_copy(init.at[fc], acc)

        plsc.subcore_barrier()

        issue_idx(0, 0)
        drain_idx()
        issue_main(fc, 0, 0)
        issue_idx(1, 1)

        def outer(g, carry):
            for b in range(2):
                w = 2 * g + b
                drain_idx()
                drain_main(fc)
                issue_main(fc, w + 1, 1 - b)
                compute(rows[b], wrow[b])
                pltpu.sync_copy(msg, acc.at[didx[b]], add=True)
                issue_idx(w + 2, b)
            return carry

        lax.fori_loop(0, nwin // 2, outer, 0)
        drain_idx()
        drain_main(fc)
        plsc.subcore_barrier()

        @pl.when(s == 0)
        def _():
            pltpu.sync_copy(acc, out.at[fc])

        plsc.subcore_barrier()


def kernel(scalar_node_features, vector_node_features, normdir, edge_index,
           edge_weight, edge_attr, Wf, bf, W1, b1, W2, b2):
    N, _, F = scalar_node_features.shape
    E = edge_index.shape[1]
    assert F == 128

    NP8 = N + 8
    epc = -(-E // (NS * 2 * WE)) * 2 * WE
    EP = epc * NS
    nwin = epc // WE
    eb = 512
    EPW = -(-(EP + 2 * WE) // eb) * eb

    xs = scalar_node_features[:, 0, :]
    xs_p = jnp.pad(xs, ((0, NP8 - N), (0, 0)))
    xv_p = jnp.pad(vector_node_features.transpose(1, 0, 2),
                   ((0, 0), (0, NP8 - N), (0, 0)))
    ea_p = jnp.pad(edge_attr, ((0, EPW - E), (0, 0)))
    ew_p = jnp.pad(edge_weight, ((0, EPW - E), (0, 0)))
    nd_p = jnp.pad(normdir, ((0, EPW - E), (0, 29)))
    src_p = jnp.pad(edge_index[0], (0, EPW - E))
    dst_p = jnp.pad(edge_index[1], (0, EPW - E),
                    constant_values=N)

    nblk = 9
    nb = NP8 // nblk
    assert NP8 % nblk == 0 and nb % 8 == 0
    xtab, init = pl.pallas_call(
        _node_tables_kernel,
        grid=(nblk,),
        in_specs=[
            pl.BlockSpec((nb, F), lambda i: (i, 0)),
            pl.BlockSpec((3, nb, F), lambda i: (0, i, 0)),
            pl.BlockSpec((F, F), lambda i: (0, 0)),
            pl.BlockSpec((F,), lambda i: (0,)),
            pl.BlockSpec((F, 3 * F), lambda i: (0, 0)),
            pl.BlockSpec((3 * F,), lambda i: (0,)),
        ],
        out_specs=[
            pl.BlockSpec((NFC, nb, 256), lambda i: (0, i, 0)),
            pl.BlockSpec((NFC, nb, 128), lambda i: (0, i, 0)),
        ],
        out_shape=[
            jax.ShapeDtypeStruct((NFC, NP8, 256), jnp.float32),
            jax.ShapeDtypeStruct((NFC, NP8, 128), jnp.float32),
        ],
    )(xs_p, xv_p, W1, b1, W2, b2)

    eblk = EPW // eb
    wmod = pl.pallas_call(
        _edge_filter_kernel,
        grid=(eblk,),
        in_specs=[
            pl.BlockSpec((eb, edge_attr.shape[1]), lambda i: (i, 0)),
            pl.BlockSpec((eb, 1), lambda i: (i, 0)),
            pl.BlockSpec((eb, 32), lambda i: (i, 0)),
            pl.BlockSpec(Wf.shape, lambda i: (0, 0)),
            pl.BlockSpec(bf.shape, lambda i: (0,)),
        ],
        out_specs=pl.BlockSpec((NFC, eb, 128), lambda i: (0, i, 0)),
        out_shape=jax.ShapeDtypeStruct((NFC, EPW, 128), jnp.float32),
    )(ea_p, ew_p, nd_p, Wf, bf)

    mesh = plsc.VectorSubcoreMesh(core_axis_name="c", subcore_axis_name="s",
                                  num_cores=NC, num_subcores=NS)
    out4 = pl.kernel(
        functools.partial(_sc_scatter_body, epc, nwin),
        out_type=jax.ShapeDtypeStruct((NFC, NP8, 128), jnp.float32),
        mesh=mesh,
        scratch_types=[
            pltpu.VMEM((WE,), jnp.int32),
            pltpu.VMEM((WE,), jnp.int32),
            pltpu.VMEM((WE,), jnp.int32),
            pltpu.VMEM((WE,), jnp.int32),
            pltpu.VMEM((WE, 256), jnp.float32),
            pltpu.VMEM((WE, 256), jnp.float32),
            pltpu.VMEM((WE, 128), jnp.float32),
            pltpu.VMEM((WE, 128), jnp.float32),
            pltpu.VMEM((WE, 128), jnp.float32),
            pltpu.VMEM_SHARED((NP8, 128), jnp.float32),
            pltpu.SemaphoreType.DMA,
            pltpu.SemaphoreType.DMA,
        ],
    )(xtab, wmod, src_p, dst_p, init)

    o = out4[:, :N, :]
    q = o[:, :, 0:32].transpose(1, 0, 2).reshape(N, 1, F)
    mu = o[:, :, 32:].reshape(NFC, N, 3, FC).transpose(1, 2, 0, 3).reshape(N, 3, F)
    return (q, mu)


# speedup vs baseline: 15.6198x; 1.2592x over previous
import functools

import jax
import jax.numpy as jnp
from jax import lax
from jax.experimental import pallas as pl
from jax.experimental.pallas import tpu as pltpu
from jax.experimental.pallas import tpu_sc as plsc

CUTOFF = 5.0
NFC = 4
FC = 32
NC, NS = 2, 16
WE = 40


def _node_tables_kernel(xs_ref, xv_ref, w1_ref, b1_ref, w2_ref, b2_ref,
                        xtab_ref, init_ref):
    xs = xs_ref[...]
    h = xs @ w1_ref[...] + b1_ref[...][None, :]
    h = h * jax.nn.sigmoid(h)
    x = h @ w2_ref[...] + b2_ref[...][None, :]
    for fc in range(NFC):
        sl = slice(fc * FC, fc * FC + FC)
        xtab_ref[fc, :, 0:32] = x[:, fc * FC:fc * FC + FC]
        xtab_ref[fc, :, 32:64] = x[:, 128 + fc * FC:128 + fc * FC + FC]
        xtab_ref[fc, :, 64:96] = x[:, 256 + fc * FC:256 + fc * FC + FC]
        init_ref[fc, :, 0:32] = xs[:, sl]
        for c in range(3):
            v = xv_ref[c, :, sl]
            xtab_ref[fc, :, 96 + 32 * c:128 + 32 * c] = v
            init_ref[fc, :, 32 + 32 * c:64 + 32 * c] = v


def _edge_filter_kernel(ea_ref, ew_ref, nd_ref, wfp_ref, bfp_ref, pnd_ref,
                        wmod_ref):
    ew = jnp.broadcast_to(ew_ref[...], (ew_ref.shape[0], 16))
    u = jnp.pi * ew / CUTOFF - (0.5 * jnp.pi)
    u2 = u * u
    sn = u * (1.0 + u2 * (-1.0 / 6.0 + u2 * (1.0 / 120.0
              + u2 * (-1.0 / 5040.0 + u2 * (1.0 / 362880.0)))))
    c = 0.5 * (1.0 - sn)
    c = c * (ew < CUTOFF).astype(jnp.float32)
    ca = ea_ref[...] * c
    w = ca @ wfp_ref[...] + c[:, 0:1] * bfp_ref[...][None, :]
    w = w + nd_ref[...] @ pnd_ref[...]
    for fc in range(NFC):
        wmod_ref[fc, :, :] = w[:, fc * 128:(fc + 1) * 128]


def _sc_scatter_body(ept, nwin,
                     xtab, wmod, srcp, dstp, init, out,
                     sidx0, sidx1, didx0, didx1,
                     rows0, rows1, wrow0, wrow1, msg, acc,
                     gsem, isem):
    c = lax.axis_index("c")
    s = lax.axis_index("s")
    sidx = (sidx0, sidx1)
    didx = (didx0, didx1)
    rows = (rows0, rows1)
    wrow = (wrow0, wrow1)

    def issue_idx(w, b):
        base = s * ept + w * WE
        pltpu.async_copy(srcp.at[pl.ds(base, WE)], sidx[b], isem)
        pltpu.async_copy(dstp.at[pl.ds(base, WE)], didx[b], isem)

    def drain_idx():
        pltpu.make_async_copy(srcp.at[pl.ds(0, WE)], sidx[0], isem).wait()
        pltpu.make_async_copy(dstp.at[pl.ds(0, WE)], didx[0], isem).wait()

    def issue_main(fc, w, b):
        base = s * ept + w * WE
        pltpu.async_copy(xtab.at[fc].at[sidx[b]], rows[b], gsem)
        pltpu.async_copy(wmod.at[fc].at[pl.ds(base, WE)], wrow[b], gsem)

    def drain_main(fc):
        pltpu.make_async_copy(xtab.at[fc].at[sidx[0]], rows[0], gsem).wait()
        pltpu.make_async_copy(wmod.at[fc].at[pl.ds(0, WE)], wrow[0], gsem).wait()

    def compute(rows_b, wrow_b):
        @plsc.parallel_loop(0, WE, unroll=4)
        def edge(e):
            nd16 = wrow_b[e, pl.ds(96, 16)]
            z16 = jnp.zeros((16,), jnp.int32)
            ndx = jnp.take_along_axis(nd16, z16, axis=0)
            ndy = jnp.take_along_axis(nd16, z16 + 1, axis=0)
            ndz = jnp.take_along_axis(nd16, z16 + 2, axis=0)
            for j in range(2):
                o = j * 16
                x0 = rows_b[e, pl.ds(0 + o, 16)]
                x1 = rows_b[e, pl.ds(32 + o, 16)]
                x2 = rows_b[e, pl.ds(64 + o, 16)]
                vx = rows_b[e, pl.ds(96 + o, 16)]
                vy = rows_b[e, pl.ds(128 + o, 16)]
                vz = rows_b[e, pl.ds(160 + o, 16)]
                w0 = wrow_b[e, pl.ds(0 + o, 16)]
                w1 = wrow_b[e, pl.ds(32 + o, 16)]
                w2 = wrow_b[e, pl.ds(64 + o, 16)]
                t = w2 * x2
                sv = w1 * x1
                msg[e, pl.ds(0 + o, 16)] = w0 * x0
                msg[e, pl.ds(32 + o, 16)] = ndx * sv + t * vx
                msg[e, pl.ds(64 + o, 16)] = ndy * sv + t * vy
                msg[e, pl.ds(96 + o, 16)] = ndz * sv + t * vz

    for p in range(2):
        fc = c * 2 + p

        @pl.when(s == 0)
        def _():
            pltpu.sync_copy(init.at[fc], acc)

        plsc.subcore_barrier()

        issue_idx(0, 0)
        drain_idx()
        issue_main(fc, 0, 0)
        issue_idx(1, 1)

        def outer(g, carry):
            for b in range(2):
                w = 2 * g + b
                drain_idx()
                drain_main(fc)
                issue_main(fc, w + 1, 1 - b)
                compute(rows[b], wrow[b])
                pltpu.sync_copy(msg, acc.at[didx[b]], add=True)
                issue_idx(w + 2, b)
            return carry

        lax.fori_loop(0, nwin // 2, outer, 0)
        drain_idx()
        drain_main(fc)
        plsc.subcore_barrier()

        @pl.when(s == 0)
        def _():
            pltpu.sync_copy(acc, out.at[fc])

        plsc.subcore_barrier()


def kernel(scalar_node_features, vector_node_features, normdir, edge_index,
           edge_weight, edge_attr, Wf, bf, W1, b1, W2, b2):
    N, _, F = scalar_node_features.shape
    E = edge_index.shape[1]
    assert F == 128

    NP8 = N + 8
    epc = -(-E // (NS * 2 * WE)) * 2 * WE
    EP = epc * NS
    nwin = epc // WE
    eb = 512
    EPW = -(-(EP + 2 * WE) // eb) * eb

    xs = scalar_node_features[:, 0, :]
    xs_p = jnp.pad(xs, ((0, NP8 - N), (0, 0)))
    xv_p = jnp.pad(vector_node_features.transpose(1, 0, 2),
                   ((0, 0), (0, NP8 - N), (0, 0)))
    ea_p = jnp.pad(edge_attr, ((0, EPW - E), (0, 0)))
    ew_p = jnp.pad(edge_weight, ((0, EPW - E), (0, 0)))
    nd_p = jnp.pad(normdir, ((0, EPW - E), (0, 29)))
    src_p = jnp.pad(edge_index[0], (0, EPW - E))
    dst_p = jnp.pad(edge_index[1], (0, EPW - E),
                    constant_values=N)

    nblk = 9
    nb = NP8 // nblk
    assert NP8 % nblk == 0 and nb % 8 == 0
    xtab, init = pl.pallas_call(
        _node_tables_kernel,
        grid=(nblk,),
        in_specs=[
            pl.BlockSpec((nb, F), lambda i: (i, 0)),
            pl.BlockSpec((3, nb, F), lambda i: (0, i, 0)),
            pl.BlockSpec((F, F), lambda i: (0, 0)),
            pl.BlockSpec((F,), lambda i: (0,)),
            pl.BlockSpec((F, 3 * F), lambda i: (0, 0)),
            pl.BlockSpec((3 * F,), lambda i: (0,)),
        ],
        out_specs=[
            pl.BlockSpec((NFC, nb, 256), lambda i: (0, i, 0)),
            pl.BlockSpec((NFC, nb, 128), lambda i: (0, i, 0)),
        ],
        out_shape=[
            jax.ShapeDtypeStruct((NFC, NP8, 256), jnp.float32),
            jax.ShapeDtypeStruct((NFC, NP8, 128), jnp.float32),
        ],
    )(xs_p, xv_p, W1, b1, W2, b2)

    RBF = Wf.shape[0]
    wcol = jnp.concatenate([
        jnp.concatenate([jnp.arange(fc * FC, fc * FC + FC),
                         jnp.arange(128 + fc * FC, 128 + fc * FC + FC),
                         jnp.arange(256 + fc * FC, 256 + fc * FC + FC),
                         jnp.full((FC,), 384)])
        for fc in range(NFC)])
    wf_ext = jnp.concatenate([Wf, jnp.zeros((RBF, 1), jnp.float32)], axis=1)
    bf_ext = jnp.concatenate([bf, jnp.zeros((1,), jnp.float32)])
    wfp = wf_ext[:, wcol]
    bfp = bf_ext[wcol]
    pnd = jnp.zeros((32, 4 * 128), jnp.float32)
    eye32 = jnp.eye(32, dtype=jnp.float32)
    for fc in range(NFC):
        pnd = pnd.at[:, fc * 128 + 96:(fc + 1) * 128].set(eye32)
    eblk = EPW // eb
    wmod = pl.pallas_call(
        _edge_filter_kernel,
        grid=(eblk,),
        in_specs=[
            pl.BlockSpec((eb, edge_attr.shape[1]), lambda i: (i, 0)),
            pl.BlockSpec((eb, 1), lambda i: (i, 0)),
            pl.BlockSpec((eb, 32), lambda i: (i, 0)),
            pl.BlockSpec((RBF, 512), lambda i: (0, 0)),
            pl.BlockSpec((512,), lambda i: (0,)),
            pl.BlockSpec((32, 512), lambda i: (0, 0)),
        ],
        out_specs=pl.BlockSpec((NFC, eb, 128), lambda i: (0, i, 0)),
        out_shape=jax.ShapeDtypeStruct((NFC, EPW, 128), jnp.float32),
    )(ea_p, ew_p, nd_p, wfp, bfp, pnd)

    mesh = plsc.VectorSubcoreMesh(core_axis_name="c", subcore_axis_name="s",
                                  num_cores=NC, num_subcores=NS)
    out4 = pl.kernel(
        functools.partial(_sc_scatter_body, epc, nwin),
        out_type=jax.ShapeDtypeStruct((NFC, NP8, 128), jnp.float32),
        mesh=mesh,
        scratch_types=[
            pltpu.VMEM((WE,), jnp.int32),
            pltpu.VMEM((WE,), jnp.int32),
            pltpu.VMEM((WE,), jnp.int32),
            pltpu.VMEM((WE,), jnp.int32),
            pltpu.VMEM((WE, 256), jnp.float32),
            pltpu.VMEM((WE, 256), jnp.float32),
            pltpu.VMEM((WE, 128), jnp.float32),
            pltpu.VMEM((WE, 128), jnp.float32),
            pltpu.VMEM((WE, 128), jnp.float32),
            pltpu.VMEM_SHARED((NP8, 128), jnp.float32),
            pltpu.SemaphoreType.DMA,
            pltpu.SemaphoreType.DMA,
        ],
    )(xtab, wmod, src_p, dst_p, init)

    o = out4[:, :N, :]
    q = o[:, :, 0:32].transpose(1, 0, 2).reshape(N, 1, F)
    mu = o[:, :, 32:].reshape(NFC, N, 3, FC).transpose(1, 2, 0, 3).reshape(N, 3, F)
    return (q, mu)


# speedup vs baseline: 16.8306x; 1.0775x over previous
import functools

import jax
import jax.numpy as jnp
import numpy as np
from jax import lax
from jax.experimental import pallas as pl
from jax.experimental.pallas import tpu as pltpu
from jax.experimental.pallas import tpu_sc as plsc

CUTOFF = 5.0
NFC = 4
FC = 32
NC, NS = 2, 16
WE = 64


def _pack_halves(x):
    k = x.shape[1] // 2
    lo = lax.bitcast_convert_type(x[:, :k].astype(jnp.bfloat16),
                                  jnp.uint16).astype(jnp.uint32)
    hi = lax.bitcast_convert_type(x[:, k:].astype(jnp.bfloat16),
                                  jnp.uint16).astype(jnp.uint32)
    return lax.bitcast_convert_type(lo | (hi << 16), jnp.float32)


def _node_tables_kernel(xs_ref, xv_ref, xvi_ref, w1_ref, b1_ref, w2_ref,
                        b2_ref, xtab_ref, init_ref):
    xs = xs_ref[...]
    h = xs @ w1_ref[...] + b1_ref[...][None, :]
    h = h * jax.nn.sigmoid(h)
    x = h @ w2_ref[...] + b2_ref[...][None, :]
    xpk = _pack_halves(x)
    vpk = [_pack_halves(xvi_ref[c, :, :]) for c in range(3)]
    for fc in range(NFC):
        sl = slice(fc * FC, fc * FC + FC)
        init_ref[fc, :, 0:32] = xs[:, sl]
        for c in range(3):
            init_ref[fc, :, 32 + 32 * c:64 + 32 * c] = xv_ref[c, :, sl]
        xtab_ref[fc, :, 0:16] = xpk[:, 16 * fc:16 * fc + 16]
        xtab_ref[fc, :, 16:32] = xpk[:, 64 + 16 * fc:64 + 16 * fc + 16]
        xtab_ref[fc, :, 32:48] = xpk[:, 128 + 16 * fc:128 + 16 * fc + 16]
        for c in range(3):
            xtab_ref[fc, :, 48 + 16 * c:64 + 16 * c] = (
                vpk[c][:, 16 * fc:16 * fc + 16])


def _edge_filter_kernel(ea_ref, ew_ref, nd_ref, wfp_ref, bfp_ref, pnd_ref,
                        wmod_ref):
    ew = jnp.broadcast_to(ew_ref[...], (ew_ref.shape[0], 16))
    u = jnp.pi * ew / CUTOFF - (0.5 * jnp.pi)
    u2 = u * u
    sn = u * (1.0 + u2 * (-1.0 / 6.0 + u2 * (1.0 / 120.0
              + u2 * (-1.0 / 5040.0 + u2 * (1.0 / 362880.0)))))
    c = 0.5 * (1.0 - sn)
    c = c * (ew < CUTOFF).astype(jnp.float32)
    ca = ea_ref[...] * c
    w = ca @ wfp_ref[...] + c[:, 0:1] * bfp_ref[...][None, :]
    w = w + nd_ref[...] @ pnd_ref[...]
    wpk = _pack_halves(w)
    for fc in range(NFC):
        wmod_ref[fc, :, :] = wpk[:, 64 * fc:64 * fc + 64]


def _sc_scatter_body(ept, nwin,
                     xtab, wmod, srcp, dstp, init, out,
                     sidx0, sidx1, didx0, didx1,
                     rows0, rows1, wrow0, wrow1, msg, acc,
                     gsem, isem):
    c = lax.axis_index("c")
    s = lax.axis_index("s")
    sidx = (sidx0, sidx1)
    didx = (didx0, didx1)
    rows = (rows0, rows1)
    wrow = (wrow0, wrow1)

    def issue_idx(w, b):
        base = pl.multiple_of(s * ept + w * WE, 8)
        pltpu.async_copy(srcp.at[pl.ds(base, WE)], sidx[b], isem)
        pltpu.async_copy(dstp.at[pl.ds(base, WE)], didx[b], isem)

    def drain_idx():
        pltpu.make_async_copy(srcp.at[pl.ds(0, WE)], sidx[0], isem).wait()
        pltpu.make_async_copy(dstp.at[pl.ds(0, WE)], didx[0], isem).wait()

    def issue_main(fc, w, b):
        base = pl.multiple_of(s * ept + w * WE, 8)
        pltpu.async_copy(xtab.at[fc].at[sidx[b]], rows[b], gsem)
        pltpu.async_copy(wmod.at[fc].at[pl.ds(base, WE)], wrow[b], gsem)

    def drain_main(fc):
        pltpu.make_async_copy(xtab.at[fc].at[sidx[0]], rows[0], gsem).wait()
        pltpu.make_async_copy(wmod.at[fc].at[pl.ds(0, WE)], wrow[0],
                              gsem).wait()

    unp = functools.partial(plsc.unpack, format=plsc.PackFormat.INTERLEAVED)

    def compute(rows_b, wrow_b):
        @plsc.parallel_loop(0, WE, unroll=2)
        def edge(e):
            z16 = jnp.zeros((16,), jnp.int32)

            def lw(g):
                v = wrow_b[e, pl.ds(16 * g, 16)]
                return unp(plsc.bitcast(v, jnp.bfloat16))

            def lr(g):
                v = rows_b[e, pl.ds(16 * g, 16)]
                return unp(plsc.bitcast(v, jnp.bfloat16))

            w0, w1, w2 = lw(0), lw(1), lw(2)
            ndv = lw(3)[0]
            ndx = jnp.take_along_axis(ndv, z16, axis=0)
            ndy = jnp.take_along_axis(ndv, z16 + 1, axis=0)
            ndz = jnp.take_along_axis(ndv, z16 + 2, axis=0)
            x0, x1, x2, vx, vy, vz = lr(0), lr(1), lr(2), lr(3), lr(4), lr(5)
            for j in range(2):
                o = j * 16
                t = w2[j] * x2[j]
                sv = w1[j] * x1[j]
                msg[e, pl.ds(0 + o, 16)] = w0[j] * x0[j]
                msg[e, pl.ds(32 + o, 16)] = ndx * sv + t * vx[j]
                msg[e, pl.ds(64 + o, 16)] = ndy * sv + t * vy[j]
                msg[e, pl.ds(96 + o, 16)] = ndz * sv + t * vz[j]

    for p in range(2):
        fc = c * 2 + p

        @pl.when(s == 0)
        def _():
            pltpu.sync_copy(init.at[fc], acc)

        plsc.subcore_barrier()

        issue_idx(0, 0)
        drain_idx()
        issue_main(fc, 0, 0)
        issue_idx(1, 1)

        def outer(g, carry):
            for b in range(2):
                w = 2 * g + b
                drain_idx()
                drain_main(fc)
                issue_main(fc, w + 1, 1 - b)
                compute(rows[b], wrow[b])
                pltpu.sync_copy(msg, acc.at[didx[b]], add=True)
                issue_idx(w + 2, b)
            return carry

        lax.fori_loop(0, nwin // 2, outer, 0)
        drain_idx()
        drain_main(fc)
        plsc.subcore_barrier()

        @pl.when(s == 0)
        def _():
            pltpu.sync_copy(acc, out.at[fc])

        plsc.subcore_barrier()


def kernel(scalar_node_features, vector_node_features, normdir, edge_index,
           edge_weight, edge_attr, Wf, bf, W1, b1, W2, b2):
    N, _, F = scalar_node_features.shape
    E = edge_index.shape[1]
    assert F == 128

    NP8 = N + 8
    epc = -(-E // (NS * 2 * WE)) * 2 * WE
    EP = epc * NS
    nwin = epc // WE
    eb = 512
    EPW = -(-(EP + 2 * WE) // eb) * eb

    lo_cols = (np.arange(NFC)[:, None] * FC + np.arange(16)[None, :]).reshape(64)
    lo3 = (np.arange(3)[:, None] * 128 + lo_cols[None, :]).reshape(192)
    hl3 = np.concatenate([lo3, lo3 + 16])
    hl1 = np.concatenate([lo_cols, lo_cols + 16])

    xs = scalar_node_features[:, 0, :]
    xs_p = jnp.pad(xs, ((0, NP8 - N), (0, 0)))
    xv_p = jnp.pad(vector_node_features.transpose(1, 0, 2),
                   ((0, 0), (0, NP8 - N), (0, 0)))
    xv_hl = xv_p[:, :, hl1]
    ea_p = jnp.pad(edge_attr, ((0, EPW - E), (0, 0)))
    ew_p = jnp.pad(edge_weight, ((0, EPW - E), (0, 0)))
    nd_p = jnp.pad(normdir, ((0, EPW - E), (0, 29)))
    src_p = jnp.pad(edge_index[0], (0, EPW - E))
    dst_p = jnp.pad(edge_index[1], (0, EPW - E),
                    constant_values=N)
    W2p = W2[:, hl3]
    b2p = b2[hl3]

    nblk = 9
    nb = NP8 // nblk
    assert NP8 % nblk == 0 and nb % 8 == 0
    xtab, init = pl.pallas_call(
        _node_tables_kernel,
        grid=(nblk,),
        in_specs=[
            pl.BlockSpec((nb, F), lambda i: (i, 0)),
            pl.BlockSpec((3, nb, F), lambda i: (0, i, 0)),
            pl.BlockSpec((3, nb, F), lambda i: (0, i, 0)),
            pl.BlockSpec((F, F), lambda i: (0, 0)),
            pl.BlockSpec((F,), lambda i: (0,)),
            pl.BlockSpec((F, 3 * F), lambda i: (0, 0)),
            pl.BlockSpec((3 * F,), lambda i: (0,)),
        ],
        out_specs=[
            pl.BlockSpec((NFC, nb, 128), lambda i: (0, i, 0)),
            pl.BlockSpec((NFC, nb, 128), lambda i: (0, i, 0)),
        ],
        out_shape=[
            jax.ShapeDtypeStruct((NFC, NP8, 128), jnp.float32),
            jax.ShapeDtypeStruct((NFC, NP8, 128), jnp.float32),
        ],
    )(xs_p, xv_p, xv_hl, W1, b1, W2p, b2p)

    RBF = Wf.shape[0]
    half = np.arange(16)
    wcol_lo = np.concatenate([
        np.concatenate([fc * FC + half, 128 + fc * FC + half,
                        256 + fc * FC + half,
                        np.full((16,), 384)])
        for fc in range(NFC)])
    wcol = np.concatenate([wcol_lo, np.where(wcol_lo == 384, 384,
                                             wcol_lo + 16)])
    wf_ext = jnp.concatenate([Wf, jnp.zeros((RBF, 1), jnp.float32)], axis=1)
    bf_ext = jnp.concatenate([bf, jnp.zeros((1,), jnp.float32)])
    wfp = wf_ext[:, wcol]
    bfp = bf_ext[wcol]
    pnd = jnp.zeros((32, 4 * 128), jnp.float32)
    eye_lo = jnp.eye(32, 16, dtype=jnp.float32)
    for fc in range(NFC):
        pnd = pnd.at[:, fc * 64 + 48:fc * 64 + 64].set(eye_lo)
    eblk = EPW // eb
    wmod = pl.pallas_call(
        _edge_filter_kernel,
        grid=(eblk,),
        in_specs=[
            pl.BlockSpec((eb, edge_attr.shape[1]), lambda i: (i, 0)),
            pl.BlockSpec((eb, 1), lambda i: (i, 0)),
            pl.BlockSpec((eb, 32), lambda i: (i, 0)),
            pl.BlockSpec((RBF, 512), lambda i: (0, 0)),
            pl.BlockSpec((512,), lambda i: (0,)),
            pl.BlockSpec((32, 512), lambda i: (0, 0)),
        ],
        out_specs=pl.BlockSpec((NFC, eb, 64), lambda i: (0, i, 0)),
        out_shape=jax.ShapeDtypeStruct((NFC, EPW, 64), jnp.float32),
    )(ea_p, ew_p, nd_p, wfp, bfp, pnd)

    mesh = plsc.VectorSubcoreMesh(core_axis_name="c", subcore_axis_name="s",
                                  num_cores=NC, num_subcores=NS)
    out4 = pl.kernel(
        functools.partial(_sc_scatter_body, epc, nwin),
        out_type=jax.ShapeDtypeStruct((NFC, NP8, 128), jnp.float32),
        mesh=mesh,
        compiler_params=pltpu.CompilerParams(needs_layout_passes=False),
        scratch_types=[
            pltpu.VMEM((WE,), jnp.int32),
            pltpu.VMEM((WE,), jnp.int32),
            pltpu.VMEM((WE,), jnp.int32),
            pltpu.VMEM((WE,), jnp.int32),
            pltpu.VMEM((WE, 128), jnp.float32),
            pltpu.VMEM((WE, 128), jnp.float32),
            pltpu.VMEM((WE, 64), jnp.float32),
            pltpu.VMEM((WE, 64), jnp.float32),
            pltpu.VMEM((WE, 128), jnp.float32),
            pltpu.VMEM_SHARED((NP8, 128), jnp.float32),
            pltpu.SemaphoreType.DMA,
            pltpu.SemaphoreType.DMA,
        ],
    )(xtab, wmod, src_p, dst_p, init)

    o = out4[:, :N, :]
    q = o[:, :, 0:32].transpose(1, 0, 2).reshape(N, 1, F)
    mu = o[:, :, 32:].reshape(NFC, N, 3, FC).transpose(1, 2, 0, 3).reshape(N, 3, F)
    return (q, mu)


# speedup vs baseline: 19.5704x; 1.1628x over previous
import functools

import jax
import jax.numpy as jnp
import numpy as np
from jax import lax
from jax.experimental import pallas as pl
from jax.experimental.pallas import tpu as pltpu
from jax.experimental.pallas import tpu_sc as plsc

CUTOFF = 5.0
NFC = 4
FC = 32
NC, NS = 2, 16
WE = 64


def _pack_halves(x):
    k = x.shape[1] // 2
    lo = lax.bitcast_convert_type(x[:, :k].astype(jnp.bfloat16),
                                  jnp.uint16).astype(jnp.uint32)
    hi = lax.bitcast_convert_type(x[:, k:].astype(jnp.bfloat16),
                                  jnp.uint16).astype(jnp.uint32)
    return lax.bitcast_convert_type(lo | (hi << 16), jnp.float32)


def _node_tables_kernel(xs_ref, xv_ref, xvi_ref, w1_ref, b1_ref, w2_ref,
                        b2_ref, xtab_ref, init_ref):
    xs = xs_ref[...]
    h = xs @ w1_ref[...] + b1_ref[...][None, :]
    h = h * jax.nn.sigmoid(h)
    x = h @ w2_ref[...] + b2_ref[...][None, :]
    xpk = _pack_halves(x)
    vpk = [_pack_halves(xvi_ref[c, :, :]) for c in range(3)]
    for fc in range(NFC):
        sl = slice(fc * FC, fc * FC + FC)
        init_ref[fc, :, 0:32] = xs[:, sl]
        for c in range(3):
            init_ref[fc, :, 32 + 32 * c:64 + 32 * c] = xv_ref[c, :, sl]
        xtab_ref[fc, :, 0:16] = xpk[:, 16 * fc:16 * fc + 16]
        xtab_ref[fc, :, 16:32] = xpk[:, 64 + 16 * fc:64 + 16 * fc + 16]
        xtab_ref[fc, :, 32:48] = xpk[:, 128 + 16 * fc:128 + 16 * fc + 16]
        for c in range(3):
            xtab_ref[fc, :, 48 + 16 * c:64 + 16 * c] = (
                vpk[c][:, 16 * fc:16 * fc + 16])


def _edge_filter_kernel(ea_ref, ew_ref, nd_ref, wfp_ref, bfp_ref, pnd_ref,
                        wmod_ref):
    ew = jnp.broadcast_to(ew_ref[...], (ew_ref.shape[0], 16))
    u = jnp.pi * ew / CUTOFF - (0.5 * jnp.pi)
    u2 = u * u
    sn = u * (1.0 + u2 * (-1.0 / 6.0 + u2 * (1.0 / 120.0
              + u2 * (-1.0 / 5040.0 + u2 * (1.0 / 362880.0)))))
    c = 0.5 * (1.0 - sn)
    c = c * (ew < CUTOFF).astype(jnp.float32)
    ca = ea_ref[...] * c
    w = ca @ wfp_ref[...] + c[:, 0:1] * bfp_ref[...][None, :]
    w = w + nd_ref[...] @ pnd_ref[...]
    wpk = _pack_halves(w)
    for cp in range(2):
        wmod_ref[cp, :, :] = wpk[:, 128 * cp:128 * cp + 128]


def _sc_scatter_body(ept, nwin,
                     xtab, wmod, srcp, dstp, init, out,
                     sidx0, sidx1, didx0, didx1,
                     rows0, rows1, wrow0, wrow1, msg, acc,
                     gsem, isem):
    c = lax.axis_index("c")
    s = lax.axis_index("s")
    sidx = (sidx0, sidx1)
    didx = (didx0, didx1)
    rows = (rows0, rows1)
    wrow = (wrow0, wrow1)

    def issue_idx(w, b):
        base = pl.multiple_of(s * ept + w * WE, 8)
        pltpu.async_copy(srcp.at[pl.ds(base, WE)], sidx[b], isem)
        pltpu.async_copy(dstp.at[pl.ds(base, WE)], didx[b], isem)

    def drain_idx():
        pltpu.make_async_copy(srcp.at[pl.ds(0, WE)], sidx[0], isem).wait()
        pltpu.make_async_copy(dstp.at[pl.ds(0, WE)], didx[0], isem).wait()

    def issue_main(fc, p, w, b):
        base = pl.multiple_of(s * ept + w * WE, 8)
        pltpu.async_copy(xtab.at[fc].at[sidx[b]], rows[b], gsem)
        pltpu.async_copy(wmod.at[c].at[pl.ds(base, WE)], wrow[b], gsem)

    def drain_main(fc, p):
        pltpu.make_async_copy(xtab.at[fc].at[sidx[0]], rows[0], gsem).wait()
        pltpu.make_async_copy(wmod.at[c].at[pl.ds(0, WE)], wrow[0],
                              gsem).wait()

    unp = functools.partial(plsc.unpack, format=plsc.PackFormat.INTERLEAVED)

    def compute(p, rows_b, wrow_b):
        @plsc.parallel_loop(0, WE, unroll=2)
        def edge(e):
            z16 = jnp.zeros((16,), jnp.int32)

            def lw(g):
                v = wrow_b[e, pl.ds(64 * p + 16 * g, 16)]
                return unp(plsc.bitcast(v, jnp.bfloat16))

            def lr(g):
                v = rows_b[e, pl.ds(16 * g, 16)]
                return unp(plsc.bitcast(v, jnp.bfloat16))

            w0, w1, w2 = lw(0), lw(1), lw(2)
            ndv = lw(3)[0]
            ndx = jnp.take_along_axis(ndv, z16, axis=0)
            ndy = jnp.take_along_axis(ndv, z16 + 1, axis=0)
            ndz = jnp.take_along_axis(ndv, z16 + 2, axis=0)
            x0, x1, x2, vx, vy, vz = lr(0), lr(1), lr(2), lr(3), lr(4), lr(5)
            for j in range(2):
                o = j * 16
                t = w2[j] * x2[j]
                sv = w1[j] * x1[j]
                msg[e, pl.ds(0 + o, 16)] = w0[j] * x0[j]
                msg[e, pl.ds(32 + o, 16)] = ndx * sv + t * vx[j]
                msg[e, pl.ds(64 + o, 16)] = ndy * sv + t * vy[j]
                msg[e, pl.ds(96 + o, 16)] = ndz * sv + t * vz[j]

    for p in range(2):
        fc = c * 2 + p

        @pl.when(s == 0)
        def _():
            pltpu.sync_copy(init.at[fc], acc)

        plsc.subcore_barrier()

        issue_idx(0, 0)
        drain_idx()
        issue_main(fc, p, 0, 0)
        issue_idx(1, 1)

        def outer(g, carry):
            for b in range(2):
                w = 2 * g + b
                drain_idx()
                drain_main(fc, p)
                issue_main(fc, p, w + 1, 1 - b)
                compute(p, rows[b], wrow[b])
                pltpu.sync_copy(msg, acc.at[didx[b]], add=True)
                issue_idx(w + 2, b)
            return carry

        lax.fori_loop(0, nwin // 2, outer, 0)
        drain_idx()
        drain_main(fc, p)
        plsc.subcore_barrier()

        @pl.when(s == 0)
        def _():
            pltpu.sync_copy(acc, out.at[fc])

        plsc.subcore_barrier()


def kernel(scalar_node_features, vector_node_features, normdir, edge_index,
           edge_weight, edge_attr, Wf, bf, W1, b1, W2, b2):
    N, _, F = scalar_node_features.shape
    E = edge_index.shape[1]
    assert F == 128

    NP8 = N + 8
    epc = -(-E // (NS * 2 * WE)) * 2 * WE
    EP = epc * NS
    nwin = epc // WE
    eb = 512
    EPW = -(-(EP + 2 * WE) // eb) * eb

    lo_cols = (np.arange(NFC)[:, None] * FC + np.arange(16)[None, :]).reshape(64)
    lo3 = (np.arange(3)[:, None] * 128 + lo_cols[None, :]).reshape(192)
    hl3 = np.concatenate([lo3, lo3 + 16])
    hl1 = np.concatenate([lo_cols, lo_cols + 16])

    xs = scalar_node_features[:, 0, :]
    xs_p = jnp.pad(xs, ((0, NP8 - N), (0, 0)))
    xv_p = jnp.pad(vector_node_features.transpose(1, 0, 2),
                   ((0, 0), (0, NP8 - N), (0, 0)))
    xv_hl = xv_p[:, :, hl1]
    src_p = jnp.pad(edge_index[0], (0, EPW - E))
    dst_p = jnp.pad(edge_index[1], (0, EPW - E),
                    constant_values=N)
    W2p = W2[:, hl3]
    b2p = b2[hl3]

    nblk = 9
    nb = NP8 // nblk
    assert NP8 % nblk == 0 and nb % 8 == 0
    xtab, init = pl.pallas_call(
        _node_tables_kernel,
        grid=(nblk,),
        in_specs=[
            pl.BlockSpec((nb, F), lambda i: (i, 0)),
            pl.BlockSpec((3, nb, F), lambda i: (0, i, 0)),
            pl.BlockSpec((3, nb, F), lambda i: (0, i, 0)),
            pl.BlockSpec((F, F), lambda i: (0, 0)),
            pl.BlockSpec((F,), lambda i: (0,)),
            pl.BlockSpec((F, 3 * F), lambda i: (0, 0)),
            pl.BlockSpec((3 * F,), lambda i: (0,)),
        ],
        out_specs=[
            pl.BlockSpec((NFC, nb, 128), lambda i: (0, i, 0)),
            pl.BlockSpec((NFC, nb, 128), lambda i: (0, i, 0)),
        ],
        out_shape=[
            jax.ShapeDtypeStruct((NFC, NP8, 128), jnp.float32),
            jax.ShapeDtypeStruct((NFC, NP8, 128), jnp.float32),
        ],
    )(xs_p, xv_p, xv_hl, W1, b1, W2p, b2p)

    RBF = Wf.shape[0]
    half = np.arange(16)
    wcol_lo = np.concatenate([
        np.concatenate([fc * FC + half, 128 + fc * FC + half,
                        256 + fc * FC + half,
                        np.full((16,), 384)])
        for fc in range(NFC)])
    wcol = np.concatenate([wcol_lo, np.where(wcol_lo == 384, 384,
                                             wcol_lo + 16)])
    wf_ext = jnp.concatenate([Wf, jnp.zeros((RBF, 1), jnp.float32)], axis=1)
    bf_ext = jnp.concatenate([bf, jnp.zeros((1,), jnp.float32)])
    wfp = wf_ext[:, wcol]
    bfp = bf_ext[wcol]
    pnd = jnp.zeros((3, 4 * 128), jnp.float32)
    for fc in range(NFC):
        pnd = pnd.at[:, fc * 64 + 48:fc * 64 + 51].set(jnp.eye(3))
    eblk = EPW // eb
    assert E % eb == 0
    clamp = E // eb - 1
    wmod = pl.pallas_call(
        _edge_filter_kernel,
        grid=(eblk,),
        in_specs=[
            pl.BlockSpec((eb, edge_attr.shape[1]),
                         lambda i: (jnp.minimum(i, clamp), 0)),
            pl.BlockSpec((eb, 1), lambda i: (jnp.minimum(i, clamp), 0)),
            pl.BlockSpec((eb, 3), lambda i: (jnp.minimum(i, clamp), 0)),
            pl.BlockSpec((RBF, 512), lambda i: (0, 0)),
            pl.BlockSpec((512,), lambda i: (0,)),
            pl.BlockSpec((3, 512), lambda i: (0, 0)),
        ],
        out_specs=pl.BlockSpec((2, eb, 128), lambda i: (0, i, 0)),
        out_shape=jax.ShapeDtypeStruct((2, EPW, 128), jnp.float32),
    )(edge_attr, edge_weight, normdir, wfp, bfp, pnd)

    mesh = plsc.VectorSubcoreMesh(core_axis_name="c", subcore_axis_name="s",
                                  num_cores=NC, num_subcores=NS)
    out4 = pl.kernel(
        functools.partial(_sc_scatter_body, epc, nwin),
        out_type=jax.ShapeDtypeStruct((NFC, NP8, 128), jnp.float32),
        mesh=mesh,
        compiler_params=pltpu.CompilerParams(needs_layout_passes=False),
        scratch_types=[
            pltpu.VMEM((WE,), jnp.int32),
            pltpu.VMEM((WE,), jnp.int32),
            pltpu.VMEM((WE,), jnp.int32),
            pltpu.VMEM((WE,), jnp.int32),
            pltpu.VMEM((WE, 128), jnp.float32),
            pltpu.VMEM((WE, 128), jnp.float32),
            pltpu.VMEM((WE, 128), jnp.float32),
            pltpu.VMEM((WE, 128), jnp.float32),
            pltpu.VMEM((WE, 128), jnp.float32),
            pltpu.VMEM_SHARED((NP8, 128), jnp.float32),
            pltpu.SemaphoreType.DMA,
            pltpu.SemaphoreType.DMA,
        ],
    )(xtab, wmod, src_p, dst_p, init)

    o = out4[:, :N, :]
    q = o[:, :, 0:32].transpose(1, 0, 2).reshape(N, 1, F)
    mu = o[:, :, 32:].reshape(NFC, N, 3, FC).transpose(1, 2, 0, 3).reshape(N, 3, F)
    return (q, mu)


# speedup vs baseline: 20.1541x; 1.0298x over previous
import functools

import jax
import jax.numpy as jnp
import numpy as np
from jax import lax
from jax.experimental import pallas as pl
from jax.experimental.pallas import tpu as pltpu
from jax.experimental.pallas import tpu_sc as plsc

CUTOFF = 5.0
NFC = 4
FC = 32
NC, NS = 2, 16
WE = 64


def _pack_halves(x):
    k = x.shape[1] // 2
    lo = lax.bitcast_convert_type(x[:, :k].astype(jnp.bfloat16),
                                  jnp.uint16).astype(jnp.uint32)
    hi = lax.bitcast_convert_type(x[:, k:].astype(jnp.bfloat16),
                                  jnp.uint16).astype(jnp.uint32)
    return lax.bitcast_convert_type(lo | (hi << 16), jnp.float32)


def _node_tables_kernel(xs_ref, xv_ref, xvi_ref, w1_ref, b1_ref, w2_ref,
                        b2_ref, xtab_ref, init_ref):
    xs = xs_ref[...]
    h = xs @ w1_ref[...] + b1_ref[...][None, :]
    h = h * jax.nn.sigmoid(h)
    x = h @ w2_ref[...] + b2_ref[...][None, :]
    xpk = _pack_halves(x)
    vpk = [_pack_halves(xvi_ref[c, :, :]) for c in range(3)]
    for fc in range(NFC):
        sl = slice(fc * FC, fc * FC + FC)
        init_ref[fc, :, 0:32] = xs[:, sl]
        for c in range(3):
            init_ref[fc, :, 32 + 32 * c:64 + 32 * c] = xv_ref[c, :, sl]
        xtab_ref[fc, :, 0:16] = xpk[:, 16 * fc:16 * fc + 16]
        xtab_ref[fc, :, 16:32] = xpk[:, 64 + 16 * fc:64 + 16 * fc + 16]
        xtab_ref[fc, :, 32:48] = xpk[:, 128 + 16 * fc:128 + 16 * fc + 16]
        for c in range(3):
            xtab_ref[fc, :, 48 + 16 * c:64 + 16 * c] = (
                vpk[c][:, 16 * fc:16 * fc + 16])


def _edge_filter_kernel(ea_ref, ew_ref, nd_ref, wfp_ref, bfp_ref, pnd_ref,
                        wmod_ref):
    ew = jnp.broadcast_to(ew_ref[...], (ew_ref.shape[0], 16))
    u = jnp.pi * ew / CUTOFF - (0.5 * jnp.pi)
    u2 = u * u
    sn = u * (1.0 + u2 * (-1.0 / 6.0 + u2 * (1.0 / 120.0
              + u2 * (-1.0 / 5040.0 + u2 * (1.0 / 362880.0)))))
    c = 0.5 * (1.0 - sn)
    c = c * (ew < CUTOFF).astype(jnp.float32)
    ca = ea_ref[...] * c
    w = ca @ wfp_ref[...] + c[:, 0:1] * bfp_ref[...][None, :]
    w = w + nd_ref[...] @ pnd_ref[...]
    wpk = _pack_halves(w)
    for cp in range(2):
        wmod_ref[cp, :, :] = wpk[:, 128 * cp:128 * cp + 128]


def _sc_scatter_body(ept, nwin,
                     xtab, wmod, srcp, dstp, init, out,
                     sidx0, sidx1, didx0, didx1, dsc0, dsc1,
                     rows0, rows1, wrow0, wrow1, msg, acc,
                     gsem, isem, ssem):
    c = lax.axis_index("c")
    s = lax.axis_index("s")
    sidx = (sidx0, sidx1)
    didx = (didx0, didx1)
    dsc = (dsc0, dsc1)
    rows = (rows0, rows1)
    wrow = (wrow0, wrow1)

    def issue_idx(w, b):
        base = pl.multiple_of(s * ept + w * WE, 8)
        pltpu.async_copy(srcp.at[pl.ds(base, WE)], sidx[b], isem)
        pltpu.async_copy(dstp.at[pl.ds(base, WE)], didx[b], isem)

    def drain_idx():
        pltpu.make_async_copy(srcp.at[pl.ds(0, WE)], sidx[0], isem).wait()
        pltpu.make_async_copy(dstp.at[pl.ds(0, WE)], didx[0], isem).wait()

    def issue_main(fc, p, w, b):
        base = pl.multiple_of(s * ept + w * WE, 8)
        pltpu.async_copy(xtab.at[fc].at[sidx[b]], rows[b], gsem)
        pltpu.async_copy(wmod.at[c].at[pl.ds(base, WE)], wrow[b], gsem)

    def drain_main(fc, p):
        pltpu.make_async_copy(xtab.at[fc].at[sidx[0]], rows[0], gsem).wait()
        pltpu.make_async_copy(wmod.at[c].at[pl.ds(0, WE)], wrow[0],
                              gsem).wait()

    unp = functools.partial(plsc.unpack, format=plsc.PackFormat.INTERLEAVED)

    def compute(p, rows_b, wrow_b):
        @plsc.parallel_loop(0, WE, unroll=2)
        def edge(e):
            z16 = jnp.zeros((16,), jnp.int32)

            def lw(g):
                v = wrow_b[e, pl.ds(64 * p + 16 * g, 16)]
                return unp(plsc.bitcast(v, jnp.bfloat16))

            def lr(g):
                v = rows_b[e, pl.ds(16 * g, 16)]
                return unp(plsc.bitcast(v, jnp.bfloat16))

            w0, w1, w2 = lw(0), lw(1), lw(2)
            ndv = lw(3)[0]
            ndx = jnp.take_along_axis(ndv, z16, axis=0)
            ndy = jnp.take_along_axis(ndv, z16 + 1, axis=0)
            ndz = jnp.take_along_axis(ndv, z16 + 2, axis=0)
            x0, x1, x2, vx, vy, vz = lr(0), lr(1), lr(2), lr(3), lr(4), lr(5)
            for j in range(2):
                o = j * 16
                t = w2[j] * x2[j]
                sv = w1[j] * x1[j]
                msg[e, pl.ds(0 + o, 16)] = w0[j] * x0[j]
                msg[e, pl.ds(32 + o, 16)] = ndx * sv + t * vx[j]
                msg[e, pl.ds(64 + o, 16)] = ndy * sv + t * vy[j]
                msg[e, pl.ds(96 + o, 16)] = ndz * sv + t * vz[j]

    def drain_scatter():
        pltpu.make_async_copy(wmod.at[c].at[pl.ds(0, WE)], msg, ssem).wait()

    for p in range(2):
        fc = c * 2 + p

        @pl.when(s == 0)
        def _():
            pltpu.sync_copy(init.at[fc], acc)

        plsc.subcore_barrier()

        issue_idx(0, 0)
        drain_idx()
        issue_main(fc, p, 0, 0)
        issue_idx(1, 1)

        def step(w, b, first):
            drain_idx()
            drain_main(fc, p)
            issue_main(fc, p, w + 1, 1 - b)
            if not first:
                drain_scatter()
            compute(p, rows[b], wrow[b])
            for i in range(WE // 16):
                dsc[b][pl.ds(16 * i, 16)] = didx[b][pl.ds(16 * i, 16)]
            pltpu.async_copy(msg, acc.at[dsc[b]], ssem, add=True)
            issue_idx(w + 2, b)

        step(0, 0, True)
        step(1, 1, False)

        def outer(g, carry):
            for b in range(2):
                step(2 + 2 * g + b, b, False)
            return carry

        lax.fori_loop(0, (nwin - 2) // 2, outer, 0)
        drain_idx()
        drain_main(fc, p)
        drain_scatter()
        plsc.subcore_barrier()

        @pl.when(s == 0)
        def _():
            pltpu.sync_copy(acc, out.at[fc])

        plsc.subcore_barrier()


def kernel(scalar_node_features, vector_node_features, normdir, edge_index,
           edge_weight, edge_attr, Wf, bf, W1, b1, W2, b2):
    N, _, F = scalar_node_features.shape
    E = edge_index.shape[1]
    assert F == 128

    NP8 = N + 8
    epc = -(-E // (NS * 2 * WE)) * 2 * WE
    EP = epc * NS
    nwin = epc // WE
    eb = 512
    EPW = -(-(EP + 2 * WE) // eb) * eb

    lo_cols = (np.arange(NFC)[:, None] * FC + np.arange(16)[None, :]).reshape(64)
    lo3 = (np.arange(3)[:, None] * 128 + lo_cols[None, :]).reshape(192)
    hl3 = np.concatenate([lo3, lo3 + 16])
    hl1 = np.concatenate([lo_cols, lo_cols + 16])

    xs = scalar_node_features[:, 0, :]
    xs_p = jnp.pad(xs, ((0, NP8 - N), (0, 0)))
    xv_p = jnp.pad(vector_node_features.transpose(1, 0, 2),
                   ((0, 0), (0, NP8 - N), (0, 0)))
    xv_hl = xv_p[:, :, hl1]
    src_p = jnp.pad(edge_index[0], (0, EPW - E))
    dst_p = jnp.pad(edge_index[1], (0, EPW - E),
                    constant_values=N)
    W2p = W2[:, hl3]
    b2p = b2[hl3]

    nblk = 9
    nb = NP8 // nblk
    assert NP8 % nblk == 0 and nb % 8 == 0
    xtab, init = pl.pallas_call(
        _node_tables_kernel,
        grid=(nblk,),
        in_specs=[
            pl.BlockSpec((nb, F), lambda i: (i, 0)),
            pl.BlockSpec((3, nb, F), lambda i: (0, i, 0)),
            pl.BlockSpec((3, nb, F), lambda i: (0, i, 0)),
            pl.BlockSpec((F, F), lambda i: (0, 0)),
            pl.BlockSpec((F,), lambda i: (0,)),
            pl.BlockSpec((F, 3 * F), lambda i: (0, 0)),
            pl.BlockSpec((3 * F,), lambda i: (0,)),
        ],
        out_specs=[
            pl.BlockSpec((NFC, nb, 128), lambda i: (0, i, 0)),
            pl.BlockSpec((NFC, nb, 128), lambda i: (0, i, 0)),
        ],
        out_shape=[
            jax.ShapeDtypeStruct((NFC, NP8, 128), jnp.float32),
            jax.ShapeDtypeStruct((NFC, NP8, 128), jnp.float32),
        ],
    )(xs_p, xv_p, xv_hl, W1, b1, W2p, b2p)

    RBF = Wf.shape[0]
    half = np.arange(16)
    wcol_lo = np.concatenate([
        np.concatenate([fc * FC + half, 128 + fc * FC + half,
                        256 + fc * FC + half,
                        np.full((16,), 384)])
        for fc in range(NFC)])
    wcol = np.concatenate([wcol_lo, np.where(wcol_lo == 384, 384,
                                             wcol_lo + 16)])
    wf_ext = jnp.concatenate([Wf, jnp.zeros((RBF, 1), jnp.float32)], axis=1)
    bf_ext = jnp.concatenate([bf, jnp.zeros((1,), jnp.float32)])
    wfp = wf_ext[:, wcol]
    bfp = bf_ext[wcol]
    pnd = jnp.zeros((3, 4 * 128), jnp.float32)
    for fc in range(NFC):
        pnd = pnd.at[:, fc * 64 + 48:fc * 64 + 51].set(jnp.eye(3))
    eblk = EPW // eb
    assert E % eb == 0
    clamp = E // eb - 1
    wmod = pl.pallas_call(
        _edge_filter_kernel,
        grid=(eblk,),
        in_specs=[
            pl.BlockSpec((eb, edge_attr.shape[1]),
                         lambda i: (jnp.minimum(i, clamp), 0)),
            pl.BlockSpec((eb, 1), lambda i: (jnp.minimum(i, clamp), 0)),
            pl.BlockSpec((eb, 3), lambda i: (jnp.minimum(i, clamp), 0)),
            pl.BlockSpec((RBF, 512), lambda i: (0, 0)),
            pl.BlockSpec((512,), lambda i: (0,)),
            pl.BlockSpec((3, 512), lambda i: (0, 0)),
        ],
        out_specs=pl.BlockSpec((2, eb, 128), lambda i: (0, i, 0)),
        out_shape=jax.ShapeDtypeStruct((2, EPW, 128), jnp.float32),
    )(edge_attr, edge_weight, normdir, wfp, bfp, pnd)

    mesh = plsc.VectorSubcoreMesh(core_axis_name="c", subcore_axis_name="s",
                                  num_cores=NC, num_subcores=NS)
    out4 = pl.kernel(
        functools.partial(_sc_scatter_body, epc, nwin),
        out_type=jax.ShapeDtypeStruct((NFC, NP8, 128), jnp.float32),
        mesh=mesh,
        compiler_params=pltpu.CompilerParams(needs_layout_passes=False),
        scratch_types=[
            pltpu.VMEM((WE,), jnp.int32),
            pltpu.VMEM((WE,), jnp.int32),
            pltpu.VMEM((WE,), jnp.int32),
            pltpu.VMEM((WE,), jnp.int32),
            pltpu.VMEM((WE,), jnp.int32),
            pltpu.VMEM((WE,), jnp.int32),
            pltpu.VMEM((WE, 128), jnp.float32),
            pltpu.VMEM((WE, 128), jnp.float32),
            pltpu.VMEM((WE, 128), jnp.float32),
            pltpu.VMEM((WE, 128), jnp.float32),
            pltpu.VMEM((WE, 128), jnp.float32),
            pltpu.VMEM_SHARED((NP8, 128), jnp.float32),
            pltpu.SemaphoreType.DMA,
            pltpu.SemaphoreType.DMA,
            pltpu.SemaphoreType.DMA,
        ],
    )(xtab, wmod, src_p, dst_p, init)

    o = out4[:, :N, :]
    q = o[:, :, 0:32].transpose(1, 0, 2).reshape(N, 1, F)
    mu = o[:, :, 32:].reshape(NFC, N, 3, FC).transpose(1, 2, 0, 3).reshape(N, 3, F)
    return (q, mu)
